# Initial kernel scaffold; baseline (speedup 1.0000x reference)
import jax
import jax.numpy as jnp
from jax.experimental import pallas as pl


def kernel(x, edge_index, batch, W1, b1, W2, b2, W3, b3):
    raise NotImplementedError("write your pallas kernel here")


# speedup vs baseline: 75.0809x; 75.0809x over previous
import functools

import jax
import jax.numpy as jnp
from jax import lax
from jax.experimental import pallas as pl
from jax.experimental.pallas import tpu as pltpu
from jax.experimental.pallas import tpu_sc as plsc

CH = 128
GRP = 8
NW = 32
NUM_GRAPHS = 128
HID = 64
OUT = 128
BN = 2000
IN_DIM = 4
DUP = 8


def _sc_agg_call(N, NG):
    per = NG // NW
    rem = NG - per * NW
    rps = ((N + 15) // 16 + 7) // 8 * 8
    rps_last = N - 15 * rps

    mesh = plsc.VectorSubcoreMesh(core_axis_name="c", subcore_axis_name="s")
    NC = 2

    @functools.partial(
        pl.kernel,
        mesh=mesh,
        compiler_params=pltpu.CompilerParams(use_tc_tiling_on_sc=False),
        out_type=(
            jax.ShapeDtypeStruct((N, DUP), jnp.float32),
            jax.ShapeDtypeStruct((N, DUP), jnp.float32),
        ),
        scratch_types=(
            pltpu.VMEM_SHARED((N, DUP), jnp.float32),
            pltpu.VMEM_SHARED((N, DUP), jnp.float32),
            pltpu.VMEM((GRP, CH), jnp.int32),
            pltpu.VMEM((GRP, CH), jnp.int32),
            pltpu.VMEM((GRP, CH, DUP), jnp.float32),
            pltpu.SemaphoreType.DMA,
            pltpu.SemaphoreType.DMA,
            pltpu.SemaphoreType.DMA,
        ),
    )
    def agg(x_hbm, src_hbm, dst_hbm, out0_hbm, out1_hbm,
            x_sh, acc_sh, idx_s, idx_d, rows, gsem, ssem, isem):
        c = lax.axis_index("c")
        s = lax.axis_index("s")
        w = s * NC + c
        r0 = s * rps

        @pl.when(s < 15)
        def _():
            pltpu.sync_copy(x_hbm.at[pl.ds(r0, rps)], x_sh.at[pl.ds(r0, rps)])
            pltpu.sync_copy(x_hbm.at[pl.ds(r0, rps)], acc_sh.at[pl.ds(r0, rps)])

        @pl.when(s == 15)
        def _():
            pltpu.sync_copy(x_hbm.at[pl.ds(15 * rps, rps_last)],
                            x_sh.at[pl.ds(15 * rps, rps_last)])
            pltpu.sync_copy(x_hbm.at[pl.ds(15 * rps, rps_last)],
                            acc_sh.at[pl.ds(15 * rps, rps_last)])

        plsc.subcore_barrier()

        def group(g):
            d0 = pltpu.async_copy(src_hbm.at[g], idx_s, isem)
            d1 = pltpu.async_copy(dst_hbm.at[g], idx_d, isem)
            d0.wait()
            d1.wait()
            gd = [pltpu.async_copy(x_sh.at[idx_s.at[j]], rows.at[j], gsem)
                  for j in range(GRP)]
            for d in gd:
                d.wait()
            sd = [pltpu.async_copy(rows.at[j], acc_sh.at[idx_d.at[j]], ssem,
                                   add=True)
                  for j in range(GRP)]
            for d in sd:
                d.wait()

        base = w * per

        def loop_body(i, carry):
            group(base + i)
            return carry

        lax.fori_loop(0, per, loop_body, 0)
        if rem:
            @pl.when(w < rem)
            def _():
                group(NW * per + w)

        plsc.subcore_barrier()

        def writeout(out_hbm):
            @pl.when(s < 15)
            def _():
                pltpu.sync_copy(acc_sh.at[pl.ds(r0, rps)],
                                out_hbm.at[pl.ds(r0, rps)])

            @pl.when(s == 15)
            def _():
                pltpu.sync_copy(acc_sh.at[pl.ds(15 * rps, rps_last)],
                                out_hbm.at[pl.ds(15 * rps, rps_last)])

        @pl.when(c == 0)
        def _():
            writeout(out0_hbm)

        @pl.when(c == 1)
        def _():
            writeout(out1_hbm)

    return agg


def _tc_mlp_call(N):
    NB = N // BN

    def body(p0_ref, p1_ref, x_ref, bat_ref, W1_ref, b1_ref, W2_ref, b2_ref,
             W3_ref, b3_ref, out_ref, acc_h, acc_c):
        i = pl.program_id(0)

        @pl.when(i == 0)
        def _():
            acc_h[...] = jnp.zeros_like(acc_h)
            acc_c[...] = jnp.zeros_like(acc_c)

        z = p0_ref[:, :IN_DIM] + p1_ref[:, :IN_DIM] - x_ref[...]
        h = jnp.maximum(z @ W1_ref[...] + b1_ref[...], 0.0)
        h = h @ W2_ref[...] + b2_ref[...]
        bb = bat_ref[0, 0, :]
        onehot = (bb[:, None] == lax.broadcasted_iota(
            jnp.int32, (BN, NUM_GRAPHS), 1)).astype(jnp.float32)
        acc_h[...] += lax.dot_general(onehot, h, (((0,), (0,)), ((), ())))
        acc_c[...] += lax.dot_general(
            onehot, jnp.ones((BN, 1), jnp.float32), (((0,), (0,)), ((), ())))

        @pl.when(i == NB - 1)
        def _():
            pooled = acc_h[...] / jnp.maximum(acc_c[...], 1.0)
            out_ref[...] = pooled @ W3_ref[...] + b3_ref[...]

    grid = (NB,)
    full = lambda shape: pl.BlockSpec(shape, lambda i: (0,) * len(shape))
    blocked = lambda shape: pl.BlockSpec(shape, lambda i: (i,) + (0,) * (len(shape) - 1))
    return pl.pallas_call(
        body,
        grid=grid,
        in_specs=[
            blocked((BN, DUP)),
            blocked((BN, DUP)),
            blocked((BN, IN_DIM)),
            blocked((1, 1, BN)),
            full((IN_DIM, HID)),
            full((1, HID)),
            full((HID, HID)),
            full((1, HID)),
            full((HID, OUT)),
            full((1, OUT)),
        ],
        out_specs=full((NUM_GRAPHS, OUT)),
        out_shape=jax.ShapeDtypeStruct((NUM_GRAPHS, OUT), jnp.float32),
        scratch_shapes=[
            pltpu.VMEM((NUM_GRAPHS, HID), jnp.float32),
            pltpu.VMEM((NUM_GRAPHS, 1), jnp.float32),
        ],
    )


def kernel(x, edge_index, batch, W1, b1, W2, b2, W3, b3):
    N = x.shape[0]
    E = edge_index.shape[1]
    NG = E // (GRP * CH)
    xd = jnp.concatenate([x, x], axis=1)
    src3 = edge_index[0].reshape(NG, GRP, CH)
    dst3 = edge_index[1].reshape(NG, GRP, CH)
    p0, p1 = _sc_agg_call(N, NG)(xd, src3, dst3)
    out = _tc_mlp_call(N)(
        p0, p1, x, batch.reshape(N // BN, 1, BN),
        W1, b1.reshape(1, HID), W2, b2.reshape(1, HID),
        W3, b3.reshape(1, OUT))
    return out


# speedup vs baseline: 87.2501x; 1.1621x over previous
import functools

import jax
import jax.numpy as jnp
from jax import lax
from jax.experimental import pallas as pl
from jax.experimental.pallas import tpu as pltpu
from jax.experimental.pallas import tpu_sc as plsc

CH = 128
GRP = 8
NW = 32
NUM_GRAPHS = 128
HID = 64
OUT = 128
BN = 2000
IN_DIM = 4
DUP = 8


def _sc_agg_call(N, NG):
    per = NG // NW
    rem = NG - per * NW
    rps = ((N + 15) // 16 + 7) // 8 * 8
    rps_last = N - 15 * rps

    mesh = plsc.VectorSubcoreMesh(core_axis_name="c", subcore_axis_name="s")
    NC = 2

    @functools.partial(
        pl.kernel,
        mesh=mesh,
        compiler_params=pltpu.CompilerParams(use_tc_tiling_on_sc=False),
        out_type=(
            jax.ShapeDtypeStruct((N, DUP), jnp.float32),
            jax.ShapeDtypeStruct((N, DUP), jnp.float32),
        ),
        scratch_types=(
            pltpu.VMEM_SHARED((N, DUP), jnp.float32),
            pltpu.VMEM_SHARED((N, DUP), jnp.float32),
            pltpu.VMEM((2, GRP, CH), jnp.int32),
            pltpu.VMEM((2, GRP, CH), jnp.int32),
            pltpu.VMEM((2, GRP, CH, DUP), jnp.float32),
            pltpu.SemaphoreType.DMA,
            pltpu.SemaphoreType.DMA,
            pltpu.SemaphoreType.DMA,
        ),
    )
    def agg(x_hbm, src_hbm, dst_hbm, out0_hbm, out1_hbm,
            x_sh, acc_sh, idx_s, idx_d, rows, gsem, ssem, isem):
        c = lax.axis_index("c")
        s = lax.axis_index("s")
        w = s * NC + c
        r0 = s * rps

        @pl.when(s < 15)
        def _():
            pltpu.sync_copy(x_hbm.at[pl.ds(r0, rps)], x_sh.at[pl.ds(r0, rps)])
            pltpu.sync_copy(x_hbm.at[pl.ds(r0, rps)], acc_sh.at[pl.ds(r0, rps)])

        @pl.when(s == 15)
        def _():
            pltpu.sync_copy(x_hbm.at[pl.ds(15 * rps, rps_last)],
                            x_sh.at[pl.ds(15 * rps, rps_last)])
            pltpu.sync_copy(x_hbm.at[pl.ds(15 * rps, rps_last)],
                            acc_sh.at[pl.ds(15 * rps, rps_last)])

        plsc.subcore_barrier()

        def load_idx(g, b):
            return (pltpu.async_copy(src_hbm.at[g], idx_s.at[b], isem),
                    pltpu.async_copy(dst_hbm.at[g], idx_d.at[b], isem))

        def fire_gathers(b):
            return [pltpu.async_copy(x_sh.at[idx_s.at[b, j]], rows.at[b, j],
                                     gsem)
                    for j in range(GRP)]

        def fire_scatters(b):
            return [pltpu.async_copy(rows.at[b, j], acc_sh.at[idx_d.at[b, j]],
                                     ssem, add=True)
                    for j in range(GRP)]

        def pair(g0):
            l0 = load_idx(g0, 0)
            l1 = load_idx(g0 + 1, 1)
            for d in l0:
                d.wait()
            gd0 = fire_gathers(0)
            for d in gd0:
                d.wait()
            sd0 = fire_scatters(0)
            for d in l1:
                d.wait()
            gd1 = fire_gathers(1)
            for d in gd1:
                d.wait()
            for d in sd0:
                d.wait()
            sd1 = fire_scatters(1)
            for d in sd1:
                d.wait()

        def single(g):
            l0 = load_idx(g, 0)
            for d in l0:
                d.wait()
            gd = fire_gathers(0)
            for d in gd:
                d.wait()
            sd = fire_scatters(0)
            for d in sd:
                d.wait()

        base = w * per

        def loop_body(i, carry):
            pair(base + 2 * i)
            return carry

        lax.fori_loop(0, per // 2, loop_body, 0)
        if per % 2:
            single(base + per - 1)
        if rem:
            @pl.when(w < rem)
            def _():
                single(NW * per + w)

        plsc.subcore_barrier()

        def writeout(out_hbm):
            @pl.when(s < 15)
            def _():
                pltpu.sync_copy(acc_sh.at[pl.ds(r0, rps)],
                                out_hbm.at[pl.ds(r0, rps)])

            @pl.when(s == 15)
            def _():
                pltpu.sync_copy(acc_sh.at[pl.ds(15 * rps, rps_last)],
                                out_hbm.at[pl.ds(15 * rps, rps_last)])

        @pl.when(c == 0)
        def _():
            writeout(out0_hbm)

        @pl.when(c == 1)
        def _():
            writeout(out1_hbm)

    return agg


def _tc_mlp_call(N):
    NB = N // BN

    def body(p0_ref, p1_ref, x_ref, bat_ref, W1_ref, b1_ref, W2_ref, b2_ref,
             W3_ref, b3_ref, out_ref, acc_h, acc_c):
        i = pl.program_id(0)

        @pl.when(i == 0)
        def _():
            acc_h[...] = jnp.zeros_like(acc_h)
            acc_c[...] = jnp.zeros_like(acc_c)

        z = p0_ref[:, :IN_DIM] + p1_ref[:, :IN_DIM] - x_ref[...]
        h = jnp.maximum(z @ W1_ref[...] + b1_ref[...], 0.0)
        h = h @ W2_ref[...] + b2_ref[...]
        bb = bat_ref[0, 0, :]
        onehot = (bb[:, None] == lax.broadcasted_iota(
            jnp.int32, (BN, NUM_GRAPHS), 1)).astype(jnp.float32)
        acc_h[...] += lax.dot_general(onehot, h, (((0,), (0,)), ((), ())))
        acc_c[...] += lax.dot_general(
            onehot, jnp.ones((BN, 1), jnp.float32), (((0,), (0,)), ((), ())))

        @pl.when(i == NB - 1)
        def _():
            pooled = acc_h[...] / jnp.maximum(acc_c[...], 1.0)
            out_ref[...] = pooled @ W3_ref[...] + b3_ref[...]

    grid = (NB,)
    full = lambda shape: pl.BlockSpec(shape, lambda i: (0,) * len(shape))
    blocked = lambda shape: pl.BlockSpec(shape, lambda i: (i,) + (0,) * (len(shape) - 1))
    return pl.pallas_call(
        body,
        grid=grid,
        in_specs=[
            blocked((BN, DUP)),
            blocked((BN, DUP)),
            blocked((BN, IN_DIM)),
            blocked((1, 1, BN)),
            full((IN_DIM, HID)),
            full((1, HID)),
            full((HID, HID)),
            full((1, HID)),
            full((HID, OUT)),
            full((1, OUT)),
        ],
        out_specs=full((NUM_GRAPHS, OUT)),
        out_shape=jax.ShapeDtypeStruct((NUM_GRAPHS, OUT), jnp.float32),
        scratch_shapes=[
            pltpu.VMEM((NUM_GRAPHS, HID), jnp.float32),
            pltpu.VMEM((NUM_GRAPHS, 1), jnp.float32),
        ],
    )


def kernel(x, edge_index, batch, W1, b1, W2, b2, W3, b3):
    N = x.shape[0]
    E = edge_index.shape[1]
    NG = E // (GRP * CH)
    xd = jnp.concatenate([x, x], axis=1)
    src3 = edge_index[0].reshape(NG, GRP, CH)
    dst3 = edge_index[1].reshape(NG, GRP, CH)
    p0, p1 = _sc_agg_call(N, NG)(xd, src3, dst3)
    out = _tc_mlp_call(N)(
        p0, p1, x, batch.reshape(N // BN, 1, BN),
        W1, b1.reshape(1, HID), W2, b2.reshape(1, HID),
        W3, b3.reshape(1, OUT))
    return out


# speedup vs baseline: 87.4760x; 1.0026x over previous
import functools

import jax
import jax.numpy as jnp
from jax import lax
from jax.experimental import pallas as pl
from jax.experimental.pallas import tpu as pltpu
from jax.experimental.pallas import tpu_sc as plsc

CH = 256
GRP = 4
NW = 32
NUM_GRAPHS = 128
HID = 64
OUT = 128
BN = 2000
IN_DIM = 4
DUP = 8


def _sc_agg_call(N, NG):
    per = NG // NW
    rem = NG - per * NW
    rps = ((N + 15) // 16 + 7) // 8 * 8
    rps_last = N - 15 * rps

    mesh = plsc.VectorSubcoreMesh(core_axis_name="c", subcore_axis_name="s")
    NC = 2

    @functools.partial(
        pl.kernel,
        mesh=mesh,
        compiler_params=pltpu.CompilerParams(use_tc_tiling_on_sc=False),
        out_type=(
            jax.ShapeDtypeStruct((N, DUP), jnp.float32),
            jax.ShapeDtypeStruct((N, DUP), jnp.float32),
        ),
        scratch_types=(
            pltpu.VMEM_SHARED((N, DUP), jnp.float32),
            pltpu.VMEM_SHARED((N, DUP), jnp.float32),
            pltpu.VMEM((2, GRP, CH), jnp.int32),
            pltpu.VMEM((2, GRP, CH), jnp.int32),
            pltpu.VMEM((2, GRP, CH, DUP), jnp.float32),
            pltpu.SemaphoreType.DMA,
            pltpu.SemaphoreType.DMA,
            pltpu.SemaphoreType.DMA,
        ),
    )
    def agg(x_hbm, src_hbm, dst_hbm, out0_hbm, out1_hbm,
            x_sh, acc_sh, idx_s, idx_d, rows, gsem, ssem, isem):
        c = lax.axis_index("c")
        s = lax.axis_index("s")
        w = s * NC + c
        r0 = s * rps

        @pl.when(s < 15)
        def _():
            pltpu.sync_copy(x_hbm.at[pl.ds(r0, rps)], x_sh.at[pl.ds(r0, rps)])
            pltpu.sync_copy(x_hbm.at[pl.ds(r0, rps)], acc_sh.at[pl.ds(r0, rps)])

        @pl.when(s == 15)
        def _():
            pltpu.sync_copy(x_hbm.at[pl.ds(15 * rps, rps_last)],
                            x_sh.at[pl.ds(15 * rps, rps_last)])
            pltpu.sync_copy(x_hbm.at[pl.ds(15 * rps, rps_last)],
                            acc_sh.at[pl.ds(15 * rps, rps_last)])

        plsc.subcore_barrier()

        def load_idx(g, b):
            return (pltpu.async_copy(src_hbm.at[g], idx_s.at[b], isem),
                    pltpu.async_copy(dst_hbm.at[g], idx_d.at[b], isem))

        def fire_gathers(b):
            return [pltpu.async_copy(x_sh.at[idx_s.at[b, j]], rows.at[b, j],
                                     gsem)
                    for j in range(GRP)]

        def fire_scatters(b):
            return [pltpu.async_copy(rows.at[b, j], acc_sh.at[idx_d.at[b, j]],
                                     ssem, add=True)
                    for j in range(GRP)]

        def pair(g0):
            l0 = load_idx(g0, 0)
            l1 = load_idx(g0 + 1, 1)
            for d in l0:
                d.wait()
            gd0 = fire_gathers(0)
            for d in gd0:
                d.wait()
            sd0 = fire_scatters(0)
            for d in l1:
                d.wait()
            gd1 = fire_gathers(1)
            for d in gd1:
                d.wait()
            for d in sd0:
                d.wait()
            sd1 = fire_scatters(1)
            for d in sd1:
                d.wait()

        def single(g):
            l0 = load_idx(g, 0)
            for d in l0:
                d.wait()
            gd = fire_gathers(0)
            for d in gd:
                d.wait()
            sd = fire_scatters(0)
            for d in sd:
                d.wait()

        base = w * per

        def loop_body(i, carry):
            pair(base + 2 * i)
            return carry

        lax.fori_loop(0, per // 2, loop_body, 0)
        if per % 2:
            single(base + per - 1)
        if rem:
            @pl.when(w < rem)
            def _():
                single(NW * per + w)

        plsc.subcore_barrier()

        def writeout(out_hbm):
            @pl.when(s < 15)
            def _():
                pltpu.sync_copy(acc_sh.at[pl.ds(r0, rps)],
                                out_hbm.at[pl.ds(r0, rps)])

            @pl.when(s == 15)
            def _():
                pltpu.sync_copy(acc_sh.at[pl.ds(15 * rps, rps_last)],
                                out_hbm.at[pl.ds(15 * rps, rps_last)])

        @pl.when(c == 0)
        def _():
            writeout(out0_hbm)

        @pl.when(c == 1)
        def _():
            writeout(out1_hbm)

    return agg


def _tc_mlp_call(N):
    NB = N // BN

    def body(p0_ref, p1_ref, x_ref, bat_ref, W1_ref, b1_ref, W2_ref, b2_ref,
             W3_ref, b3_ref, out_ref, acc_h, acc_c):
        i = pl.program_id(0)

        @pl.when(i == 0)
        def _():
            acc_h[...] = jnp.zeros_like(acc_h)
            acc_c[...] = jnp.zeros_like(acc_c)

        z = p0_ref[:, :IN_DIM] + p1_ref[:, :IN_DIM] - x_ref[...]
        h = jnp.maximum(z @ W1_ref[...] + b1_ref[...], 0.0)
        h = h @ W2_ref[...] + b2_ref[...]
        bb = bat_ref[0, 0, :]
        onehot = (bb[:, None] == lax.broadcasted_iota(
            jnp.int32, (BN, NUM_GRAPHS), 1)).astype(jnp.float32)
        acc_h[...] += lax.dot_general(onehot, h, (((0,), (0,)), ((), ())))
        acc_c[...] += lax.dot_general(
            onehot, jnp.ones((BN, 1), jnp.float32), (((0,), (0,)), ((), ())))

        @pl.when(i == NB - 1)
        def _():
            pooled = acc_h[...] / jnp.maximum(acc_c[...], 1.0)
            out_ref[...] = pooled @ W3_ref[...] + b3_ref[...]

    grid = (NB,)
    full = lambda shape: pl.BlockSpec(shape, lambda i: (0,) * len(shape))
    blocked = lambda shape: pl.BlockSpec(shape, lambda i: (i,) + (0,) * (len(shape) - 1))
    return pl.pallas_call(
        body,
        grid=grid,
        in_specs=[
            blocked((BN, DUP)),
            blocked((BN, DUP)),
            blocked((BN, IN_DIM)),
            blocked((1, 1, BN)),
            full((IN_DIM, HID)),
            full((1, HID)),
            full((HID, HID)),
            full((1, HID)),
            full((HID, OUT)),
            full((1, OUT)),
        ],
        out_specs=full((NUM_GRAPHS, OUT)),
        out_shape=jax.ShapeDtypeStruct((NUM_GRAPHS, OUT), jnp.float32),
        scratch_shapes=[
            pltpu.VMEM((NUM_GRAPHS, HID), jnp.float32),
            pltpu.VMEM((NUM_GRAPHS, 1), jnp.float32),
        ],
    )


def kernel(x, edge_index, batch, W1, b1, W2, b2, W3, b3):
    N = x.shape[0]
    E = edge_index.shape[1]
    NG = E // (GRP * CH)
    xd = jnp.concatenate([x, x], axis=1)
    src3 = edge_index[0].reshape(NG, GRP, CH)
    dst3 = edge_index[1].reshape(NG, GRP, CH)
    p0, p1 = _sc_agg_call(N, NG)(xd, src3, dst3)
    out = _tc_mlp_call(N)(
        p0, p1, x, batch.reshape(N // BN, 1, BN),
        W1, b1.reshape(1, HID), W2, b2.reshape(1, HID),
        W3, b3.reshape(1, OUT))
    return out
